# baseline (speedup 1.0000x reference)
import jax
import jax.numpy as jnp
from jax import lax
from jax.experimental import pallas as pl
from jax.experimental.pallas import tpu as pltpu
from jax.experimental.pallas import tpu_sc as plsc

B = 4096
EMBED = 64
NCOMP = 16
NTAB = 100000
L = 16
NW = 32
NDIMS = 2 * EMBED + 6 * NCOMP
GSTEPS = B // L


def _gather_dim(src_t, e, idx_v, row_v, out_v, out_hbm, r_flat):
    pltpu.sync_copy(src_t.at[e, :], row_v)

    def gstep(j, carry):
        iv = idx_v[pl.ds(j * L, L)]
        out_v[pl.ds(j * L, L)] = plsc.load_gather(row_v, [iv])
        return carry

    lax.fori_loop(0, GSTEPS, gstep, 0)
    pltpu.sync_copy(out_v, out_hbm.at[pl.ds(r_flat * B, B)])


def _sc_body(uid_hbm, iid_hbm, ut_t, it_t,
             gu0_t, gi0_t, gu1_t, gi1_t, gu2_t, gi2_t,
             out_hbm,
             uid_v, iid_v, row_v, out_v):
    wid = lax.axis_index("s") * 2 + lax.axis_index("c")
    pltpu.sync_copy(uid_hbm, uid_v)
    pltpu.sync_copy(iid_hbm, iid_v)

    @pl.when(wid < 16)
    def _():
        for j in range(4):
            e = wid * 4 + j
            _gather_dim(ut_t, e, uid_v, row_v, out_v, out_hbm, e)

    @pl.when(wid >= 16)
    def _():
        for j in range(4):
            e = (wid - 16) * 4 + j
            _gather_dim(it_t, e, iid_v, row_v, out_v, out_hbm, EMBED + e)

    gtabs = [(gu0_t, uid_v), (gi0_t, iid_v),
             (gu1_t, uid_v), (gi1_t, iid_v),
             (gu2_t, uid_v), (gi2_t, iid_v)]
    for t, (tab, idxv) in enumerate(gtabs):
        lo_w = max(0, -(-(NCOMP * t - 2) // 3))
        hi_w = (NCOMP * t + NCOMP - 1) // 3

        @pl.when((wid >= lo_w) & (wid <= hi_w))
        def _(t=t, tab=tab, idxv=idxv):
            for j in range(3):
                g = 3 * wid + j

                @pl.when((g >= NCOMP * t) & (g < NCOMP * (t + 1)))
                def _(g=g, t=t, tab=tab, idxv=idxv):
                    e = g - NCOMP * t
                    _gather_dim(tab, e, idxv, row_v, out_v, out_hbm,
                                2 * EMBED + g)


_sc_call = pl.kernel(
    _sc_body,
    out_type=jax.ShapeDtypeStruct((NDIMS * B,), jnp.float32),
    mesh=plsc.VectorSubcoreMesh(core_axis_name="c", subcore_axis_name="s"),
    compiler_params=pltpu.CompilerParams(
        needs_layout_passes=False, use_tc_tiling_on_sc=True),
    scratch_types=[
        pltpu.VMEM((B,), jnp.int32),
        pltpu.VMEM((B,), jnp.int32),
        pltpu.VMEM((NTAB,), jnp.float32),
        pltpu.VMEM((B,), jnp.float32),
    ],
)


KB = 512
NBLK = B // KB


def _bf(x):
    return x.astype(jnp.bfloat16).astype(jnp.float32)


def _tail_body(g_ref, w0_ref, w1_ref, w2_ref, b_ref, out_ref):
    def dim(r):
        return g_ref[pl.ds(r * B, B)]

    d0 = jnp.zeros((B,), jnp.float32)
    d1 = jnp.zeros((B,), jnp.float32)
    d2 = jnp.zeros((B,), jnp.float32)
    base = 2 * EMBED
    for c in range(NCOMP):
        d0 = d0 + dim(base + c) * dim(base + NCOMP + c)
        d1 = d1 + dim(base + 2 * NCOMP + c) * dim(base + 3 * NCOMP + c)
        d2 = d2 + dim(base + 4 * NCOMP + c) * dim(base + 5 * NCOMP + c)
    b0 = b_ref[0, 0]
    b1 = b_ref[0, 1]
    b2 = b_ref[0, 2]
    out0 = jnp.zeros((B,), jnp.float32)
    for e in range(EMBED):
        x = dim(e) * dim(EMBED + e) + d0
        out0 = out0 + _bf(x) * _bf(w0_ref[0, e])
    out0 = out0 + b0

    def layer(prev, d, wb, b):
        parts = []
        for kb in range(NBLK):
            dk = d[kb * KB:(kb + 1) * KB].reshape(KB, 1)
            m = _bf(prev.reshape(1, B) + dk)
            parts.append(jnp.sum(m * wb, axis=1) + b)
        return jnp.concatenate(parts)

    w1b = _bf(w1_ref[...]).reshape(1, B)
    w2b = _bf(w2_ref[...]).reshape(1, B)
    out1 = layer(out0, d1, w1b, b1)
    out_ref[...] = layer(out1, d2, w2b, b2)


_tail_call = pl.pallas_call(
    _tail_body,
    out_shape=jax.ShapeDtypeStruct((B,), jnp.float32),
    in_specs=[
        pl.BlockSpec(memory_space=pltpu.VMEM),
        pl.BlockSpec(memory_space=pltpu.SMEM),
        pl.BlockSpec(memory_space=pltpu.VMEM),
        pl.BlockSpec(memory_space=pltpu.VMEM),
        pl.BlockSpec(memory_space=pltpu.SMEM),
    ],
    out_specs=pl.BlockSpec(memory_space=pltpu.VMEM),
)


def kernel(user_id, item_id, user_table, item_table,
           gcn_user_0, gcn_item_0, W_0, b_0,
           gcn_user_1, gcn_item_1, W_1, b_1,
           gcn_user_2, gcn_item_2, W_2, b_2):
    uid = user_id.reshape(B).astype(jnp.int32)
    iid = item_id.reshape(B).astype(jnp.int32)
    g = _sc_call(
        uid, iid, user_table.T, item_table.T,
        gcn_user_0.T, gcn_item_0.T, gcn_user_1.T, gcn_item_1.T,
        gcn_user_2.T, gcn_item_2.T)
    b = jnp.concatenate([b_0, b_1, b_2]).reshape(1, 3)
    out = _tail_call(g, W_0.reshape(1, EMBED), W_1.reshape(B),
                     W_2.reshape(B), b)
    return out.reshape(B, 1)


# speedup vs baseline: 1.0393x; 1.0393x over previous
import jax
import jax.numpy as jnp
from jax import lax
from jax.experimental import pallas as pl
from jax.experimental.pallas import tpu as pltpu
from jax.experimental.pallas import tpu_sc as plsc

B = 4096
EMBED = 64
NCOMP = 16
NTAB = 100000
L = 16
NW = 32
NDIMS = 2 * EMBED + 6 * NCOMP
GSTEPS = B // L


def _gather_dim(src_t, e, idx_v, row_v, out_v, out_hbm, r_flat):
    pltpu.sync_copy(src_t.at[e, :], row_v)

    def gstep(j, carry):
        iv = idx_v[pl.ds(j * L, L)]
        out_v[pl.ds(j * L, L)] = plsc.load_gather(row_v, [iv])
        return carry

    lax.fori_loop(0, GSTEPS, gstep, 0)
    pltpu.sync_copy(out_v, out_hbm.at[pl.ds(r_flat * B, B)])


def _sc_body(uid_hbm, iid_hbm, ut_t, it_t,
             gu0_t, gi0_t, gu1_t, gi1_t, gu2_t, gi2_t,
             out_hbm,
             uid_v, iid_v, row_v, out_v):
    wid = lax.axis_index("s") * 2 + lax.axis_index("c")
    pltpu.sync_copy(uid_hbm, uid_v)
    pltpu.sync_copy(iid_hbm, iid_v)

    @pl.when(wid < 16)
    def _():
        for j in range(4):
            e = wid * 4 + j
            _gather_dim(ut_t, e, uid_v, row_v, out_v, out_hbm, e)

    @pl.when(wid >= 16)
    def _():
        for j in range(4):
            e = (wid - 16) * 4 + j
            _gather_dim(it_t, e, iid_v, row_v, out_v, out_hbm, EMBED + e)

    gtabs = [(gu0_t, uid_v), (gi0_t, iid_v),
             (gu1_t, uid_v), (gi1_t, iid_v),
             (gu2_t, uid_v), (gi2_t, iid_v)]
    for t, (tab, idxv) in enumerate(gtabs):
        lo_w = max(0, -(-(NCOMP * t - 2) // 3))
        hi_w = (NCOMP * t + NCOMP - 1) // 3

        @pl.when((wid >= lo_w) & (wid <= hi_w))
        def _(t=t, tab=tab, idxv=idxv):
            for j in range(3):
                g = 3 * wid + j

                @pl.when((g >= NCOMP * t) & (g < NCOMP * (t + 1)))
                def _(g=g, t=t, tab=tab, idxv=idxv):
                    e = g - NCOMP * t
                    _gather_dim(tab, e, idxv, row_v, out_v, out_hbm,
                                2 * EMBED + g)


_sc_call = pl.kernel(
    _sc_body,
    out_type=jax.ShapeDtypeStruct((NDIMS * B,), jnp.float32),
    mesh=plsc.VectorSubcoreMesh(core_axis_name="c", subcore_axis_name="s"),
    compiler_params=pltpu.CompilerParams(
        needs_layout_passes=False, use_tc_tiling_on_sc=True),
    scratch_types=[
        pltpu.VMEM((B,), jnp.int32),
        pltpu.VMEM((B,), jnp.int32),
        pltpu.VMEM((NTAB,), jnp.float32),
        pltpu.VMEM((B,), jnp.float32),
    ],
)


KB = 512
NBLK = B // KB


def _bf(x):
    return x.astype(jnp.bfloat16).astype(jnp.float32)


def _tail_body(g_ref, w0_ref, w1_ref, w2_ref, b_ref, out_ref):
    def dim(r):
        return g_ref[pl.ds(r * B, B)]

    d0 = jnp.zeros((B,), jnp.float32)
    d1 = jnp.zeros((B,), jnp.float32)
    d2 = jnp.zeros((B,), jnp.float32)
    base = 2 * EMBED
    for c in range(NCOMP):
        d0 = d0 + dim(base + c) * dim(base + NCOMP + c)
        d1 = d1 + dim(base + 2 * NCOMP + c) * dim(base + 3 * NCOMP + c)
        d2 = d2 + dim(base + 4 * NCOMP + c) * dim(base + 5 * NCOMP + c)
    b0 = b_ref[0, 0]
    b1 = b_ref[0, 1]
    b2 = b_ref[0, 2]
    out0 = jnp.zeros((B,), jnp.float32)
    for e in range(EMBED):
        x = dim(e) * dim(EMBED + e) + d0
        out0 = out0 + _bf(x) * _bf(w0_ref[0, e])
    out0 = out0 + b0

    def layer(prev, d, wcol, b):
        parts = []
        for kb in range(NBLK):
            dk = d[kb * KB:(kb + 1) * KB].reshape(KB, 1)
            m = (prev.reshape(1, B) + dk).astype(jnp.bfloat16)
            y = jax.lax.dot_general(
                m, wcol, (((1,), (0,)), ((), ())),
                preferred_element_type=jnp.float32)
            parts.append(y.reshape(KB) + b)
        return jnp.concatenate(parts)

    w1b = w1_ref[...].astype(jnp.bfloat16).reshape(B, 1)
    w2b = w2_ref[...].astype(jnp.bfloat16).reshape(B, 1)
    out1 = layer(out0, d1, w1b, b1)
    out_ref[...] = layer(out1, d2, w2b, b2)


_tail_call = pl.pallas_call(
    _tail_body,
    out_shape=jax.ShapeDtypeStruct((B,), jnp.float32),
    in_specs=[
        pl.BlockSpec(memory_space=pltpu.VMEM),
        pl.BlockSpec(memory_space=pltpu.SMEM),
        pl.BlockSpec(memory_space=pltpu.VMEM),
        pl.BlockSpec(memory_space=pltpu.VMEM),
        pl.BlockSpec(memory_space=pltpu.SMEM),
    ],
    out_specs=pl.BlockSpec(memory_space=pltpu.VMEM),
)


def kernel(user_id, item_id, user_table, item_table,
           gcn_user_0, gcn_item_0, W_0, b_0,
           gcn_user_1, gcn_item_1, W_1, b_1,
           gcn_user_2, gcn_item_2, W_2, b_2):
    uid = user_id.reshape(B).astype(jnp.int32)
    iid = item_id.reshape(B).astype(jnp.int32)
    g = _sc_call(
        uid, iid, user_table.T, item_table.T,
        gcn_user_0.T, gcn_item_0.T, gcn_user_1.T, gcn_item_1.T,
        gcn_user_2.T, gcn_item_2.T)
    b = jnp.concatenate([b_0, b_1, b_2]).reshape(1, 3)
    out = _tail_call(g, W_0.reshape(1, EMBED), W_1.reshape(B),
                     W_2.reshape(B), b)
    return out.reshape(B, 1)


# speedup vs baseline: 1.0980x; 1.0565x over previous
import jax
import jax.numpy as jnp
from jax import lax
from jax.experimental import pallas as pl
from jax.experimental.pallas import tpu as pltpu
from jax.experimental.pallas import tpu_sc as plsc

B = 4096
EMBED = 64
NCOMP = 16
NTAB = 100000
L = 16
NW = 32
GSTEPS = B // L
ND1 = 2 * EMBED + 4 * NCOMP
ND2 = 2 * NCOMP


def _gather_dim(src_t, e, idx_v, row_v, out_v, out_hbm, r_flat):
    pltpu.sync_copy(src_t.at[e, :], row_v)

    def gstep(j, carry):
        iv = idx_v[pl.ds(j * L, L)]
        out_v[pl.ds(j * L, L)] = plsc.load_gather(row_v, [iv])
        return carry

    lax.fori_loop(0, GSTEPS, gstep, 0)
    pltpu.sync_copy(out_v, out_hbm.at[pl.ds(r_flat * B, B)])


def _sc1_body(uid_hbm, iid_hbm, ut_t, it_t, gu0_t, gi0_t, gu1_t, gi1_t,
              out_hbm, uid_v, iid_v, row_v, out_v):
    wid = lax.axis_index("s") * 2 + lax.axis_index("c")
    pltpu.sync_copy(uid_hbm, uid_v)
    pltpu.sync_copy(iid_hbm, iid_v)

    @pl.when(wid < 16)
    def _():
        for j in range(4):
            e = wid * 4 + j
            _gather_dim(ut_t, e, uid_v, row_v, out_v, out_hbm, e)

    @pl.when(wid >= 16)
    def _():
        for j in range(4):
            e = (wid - 16) * 4 + j
            _gather_dim(it_t, e, iid_v, row_v, out_v, out_hbm, EMBED + e)

    for t, (tab, idxv) in enumerate(
            [(gu0_t, uid_v), (gi0_t, iid_v), (gu1_t, uid_v), (gi1_t, iid_v)]):
        @pl.when((wid >= 8 * t) & (wid < 8 * (t + 1)))
        def _(t=t, tab=tab, idxv=idxv):
            for j in range(2):
                e = 2 * (wid - 8 * t) + j
                _gather_dim(tab, e, idxv, row_v, out_v, out_hbm,
                            2 * EMBED + NCOMP * t + e)


def _sc2_body(uid_hbm, iid_hbm, gu2_t, gi2_t,
              out_hbm, uid_v, iid_v, row_v, out_v):
    wid = lax.axis_index("s") * 2 + lax.axis_index("c")
    pltpu.sync_copy(uid_hbm, uid_v)
    pltpu.sync_copy(iid_hbm, iid_v)

    @pl.when(wid < 16)
    def _():
        _gather_dim(gu2_t, wid, uid_v, row_v, out_v, out_hbm, wid)

    @pl.when(wid >= 16)
    def _():
        _gather_dim(gi2_t, wid - 16, iid_v, row_v, out_v, out_hbm, wid)


_SC_SCRATCH = [
    pltpu.VMEM((B,), jnp.int32),
    pltpu.VMEM((B,), jnp.int32),
    pltpu.VMEM((NTAB,), jnp.float32),
    pltpu.VMEM((B,), jnp.float32),
]
_SC_PARAMS = pltpu.CompilerParams(
    needs_layout_passes=False, use_tc_tiling_on_sc=True)
_MESH = plsc.VectorSubcoreMesh(core_axis_name="c", subcore_axis_name="s")

_sc1_call = pl.kernel(
    _sc1_body,
    out_type=jax.ShapeDtypeStruct((ND1 * B,), jnp.float32),
    mesh=_MESH, compiler_params=_SC_PARAMS, scratch_types=_SC_SCRATCH)

_sc2_call = pl.kernel(
    _sc2_body,
    out_type=jax.ShapeDtypeStruct((ND2 * B,), jnp.float32),
    mesh=_MESH, compiler_params=_SC_PARAMS, scratch_types=_SC_SCRATCH)


KB = 512
NBLK = B // KB


def _bf(x):
    return x.astype(jnp.bfloat16).astype(jnp.float32)


def _mxu_layer(prev, d, wcol, b):
    parts = []
    for kb in range(NBLK):
        dk = d[kb * KB:(kb + 1) * KB].reshape(KB, 1)
        m = (prev.reshape(1, B) + dk).astype(jnp.bfloat16)
        y = jax.lax.dot_general(
            m, wcol, (((1,), (0,)), ((), ())),
            preferred_element_type=jnp.float32)
        parts.append(y.reshape(KB) + b)
    return jnp.concatenate(parts)


def _tail1_body(g_ref, w0_ref, w1_ref, b_ref, out_ref):
    def dim(r):
        return g_ref[pl.ds(r * B, B)]

    base = 2 * EMBED
    d0 = jnp.zeros((B,), jnp.float32)
    d1 = jnp.zeros((B,), jnp.float32)
    for c in range(NCOMP):
        d0 = d0 + dim(base + c) * dim(base + NCOMP + c)
        d1 = d1 + dim(base + 2 * NCOMP + c) * dim(base + 3 * NCOMP + c)
    b0 = b_ref[0, 0]
    b1 = b_ref[0, 1]
    out0 = jnp.zeros((B,), jnp.float32)
    for e in range(EMBED):
        x = dim(e) * dim(EMBED + e) + d0
        out0 = out0 + _bf(x) * _bf(w0_ref[0, e])
    out0 = out0 + b0
    w1b = w1_ref[...].astype(jnp.bfloat16).reshape(B, 1)
    out_ref[...] = _mxu_layer(out0, d1, w1b, b1)


def _tail2_body(g_ref, out1_ref, w2_ref, b_ref, out_ref):
    d2 = jnp.zeros((B,), jnp.float32)
    for c in range(NCOMP):
        d2 = d2 + g_ref[pl.ds(c * B, B)] * g_ref[pl.ds((NCOMP + c) * B, B)]
    w2b = w2_ref[...].astype(jnp.bfloat16).reshape(B, 1)
    out_ref[...] = _mxu_layer(out1_ref[...], d2, w2b, b_ref[0, 2])


_tail1_call = pl.pallas_call(
    _tail1_body,
    out_shape=jax.ShapeDtypeStruct((B,), jnp.float32),
    in_specs=[
        pl.BlockSpec(memory_space=pltpu.VMEM),
        pl.BlockSpec(memory_space=pltpu.SMEM),
        pl.BlockSpec(memory_space=pltpu.VMEM),
        pl.BlockSpec(memory_space=pltpu.SMEM),
    ],
    out_specs=pl.BlockSpec(memory_space=pltpu.VMEM),
)

_tail2_call = pl.pallas_call(
    _tail2_body,
    out_shape=jax.ShapeDtypeStruct((B,), jnp.float32),
    in_specs=[
        pl.BlockSpec(memory_space=pltpu.VMEM),
        pl.BlockSpec(memory_space=pltpu.VMEM),
        pl.BlockSpec(memory_space=pltpu.VMEM),
        pl.BlockSpec(memory_space=pltpu.SMEM),
    ],
    out_specs=pl.BlockSpec(memory_space=pltpu.VMEM),
)


def kernel(user_id, item_id, user_table, item_table,
           gcn_user_0, gcn_item_0, W_0, b_0,
           gcn_user_1, gcn_item_1, W_1, b_1,
           gcn_user_2, gcn_item_2, W_2, b_2):
    uid = user_id.reshape(B).astype(jnp.int32)
    iid = item_id.reshape(B).astype(jnp.int32)
    g1 = _sc1_call(uid, iid, user_table.T, item_table.T,
                   gcn_user_0.T, gcn_item_0.T, gcn_user_1.T, gcn_item_1.T)
    g2 = _sc2_call(uid, iid, gcn_user_2.T, gcn_item_2.T)
    b = jnp.concatenate([b_0, b_1, b_2]).reshape(1, 3)
    out1 = _tail1_call(g1, W_0.reshape(1, EMBED), W_1.reshape(B), b)
    out = _tail2_call(g2, out1, W_2.reshape(B), b)
    return out.reshape(B, 1)
